# baseline (speedup 1.0000x reference)
import functools

import jax
import jax.numpy as jnp
from jax import lax
from jax.experimental import pallas as pl
from jax.experimental.pallas import tpu as pltpu
from jax.experimental.pallas import tpu_sc as plsc

N_TOKEN = 1000000
D_EMBED = 64
BATCH = 4096
HIST = 200

NC = 2
NS = 16
NW = NC * NS
IW = 128
DPAD = 128

W = 512
WR = W // 2
NROWS = N_TOKEN // 2
NBLK = N_TOKEN // W
BLK_MAIN = NBLK // NW * NW
BLKS_PER_W = BLK_MAIN // NW
TAILB = BLK_MAIN * W
TAIL0 = NBLK * W

_mesh = plsc.VectorSubcoreMesh(core_axis_name="c", subcore_axis_name="s")
_params = pltpu.CompilerParams(
    use_tc_tiling_on_sc=True, needs_layout_passes=False
)


@functools.partial(
    pl.kernel,
    mesh=_mesh,
    compiler_params=_params,
    out_type=jax.ShapeDtypeStruct((NROWS, DPAD), jnp.float32),
    scratch_types=[
        pltpu.VMEM((2, D_EMBED, W), jnp.float32),
        pltpu.VMEM((2, WR, DPAD), jnp.float32),
        pltpu.SemaphoreType.DMA((2,)),
        pltpu.SemaphoreType.DMA((2,)),
    ],
)
def _table_fmt(table_t, tail_slab, out_hbm, sin, sout, isem, wsem):
    wid = lax.axis_index("s") * NC + lax.axis_index("c")

    lanes = lax.broadcasted_iota(jnp.int32, (16,), 0)
    rowsel = [lanes + k * 16 for k in range(D_EMBED // 16)]

    def col_of(j):
        return wid + NW * j

    def fire_fetch(j, slot):
        pltpu.async_copy(
            table_t.at[:, pl.ds(col_of(j) * W, W)], sin.at[slot],
            isem.at[slot],
        )

    def wait_fetch(slot):
        pltpu.make_async_copy(
            table_t.at[:, pl.ds(0, W)], sin.at[slot], isem.at[slot]
        ).wait()

    def transpose(slot):
        @plsc.parallel_loop(0, WR, unroll=4)
        def _(r):
            for t in range(2):
                cols = jnp.full((16,), 2 * r + t, jnp.int32)
                for k in range(D_EMBED // 16):
                    vals = plsc.load_gather(sin.at[slot], [rowsel[k], cols])
                    sout[slot, r, pl.ds(t * D_EMBED + k * 16, 16)] = vals

    def fire_wb(j, slot):
        pltpu.async_copy(
            sout.at[slot],
            out_hbm.at[pl.ds(col_of(j) * WR, WR)],
            wsem.at[slot],
        )

    def wait_wb(slot):
        pltpu.make_async_copy(
            sout.at[slot], out_hbm.at[pl.ds(0, WR)], wsem.at[slot]
        ).wait()

    @pl.when(wid == 6)
    def _():
        pltpu.sync_copy(tail_slab, out_hbm.at[pl.ds(TAIL0 // 2, 32)])

    fire_fetch(0, 0)
    fire_fetch(1, 1)

    def jbody(j, carry):
        s = j % 2
        wait_fetch(s)

        @pl.when(j >= 2)
        def _():
            wait_wb(s)

        transpose(s)
        fire_wb(j, s)

        @pl.when(j + 2 < BLKS_PER_W)
        def _():
            fire_fetch(j + 2, s)

        return carry

    lax.fori_loop(0, BLKS_PER_W, jbody, 0)
    wait_wb(0)
    wait_wb(1)

    @pl.when(wid == 5)
    def _():
        pltpu.async_copy(
            table_t.at[:, pl.ds(TAILB, W)], sin.at[0], isem.at[0]
        )
        wait_fetch(0)
        transpose(0)
        pltpu.async_copy(
            sout.at[0], out_hbm.at[pl.ds(TAILB // 2, WR)], wsem.at[0]
        )
        wait_wb(0)


@functools.partial(
    pl.kernel,
    mesh=_mesh,
    compiler_params=_params,
    out_type=jax.ShapeDtypeStruct((HIST, D_EMBED, BATCH), jnp.float32),
    scratch_types=[
        pltpu.VMEM((HIST, IW), jnp.int32),
        pltpu.VMEM((HIST, IW), jnp.int32),
        pltpu.VMEM((2, IW, DPAD), jnp.float32),
        pltpu.VMEM((2, D_EMBED, IW), jnp.float32),
        pltpu.SemaphoreType.DMA((2,)),
        pltpu.SemaphoreType.DMA((2,)),
    ],
)
def _emb_lookup(
    idx_t, table_hbm, out_hbm, idx_v, idx2_v, rows_v, slab_v, gsem, osem
):
    wid = lax.axis_index("s") * NC + lax.axis_index("c")
    b0 = wid * IW
    pltpu.sync_copy(idx_t.at[:, pl.ds(b0, IW)], idx_v)

    lanes = lax.broadcasted_iota(jnp.int32, (16,), 0)
    rowsel = [lanes + k * 16 for k in range(IW // 16)]

    @plsc.parallel_loop(0, HIST, unroll=4)
    def _(h):
        for k in range(IW // 16):
            raw = idx_v[h, pl.ds(k * 16, 16)]
            idx2_v[h, pl.ds(k * 16, 16)] = raw >> 1

    def fire_gather(h, slot):
        pltpu.async_copy(
            table_hbm.at[idx2_v.at[h]], rows_v.at[slot], gsem.at[slot]
        )

    def wait_gather(slot):
        pltpu.make_async_copy(
            table_hbm.at[pl.ds(0, IW)], rows_v.at[slot], gsem.at[slot]
        ).wait()

    def transpose(h, slot):
        offs = [
            (idx_v[h, pl.ds(k * 16, 16)] & 1) << 6
            for k in range(IW // 16)
        ]

        @plsc.parallel_loop(0, D_EMBED, unroll=4)
        def _(d):
            for k in range(IW // 16):
                vals = plsc.load_gather(
                    rows_v.at[slot], [rowsel[k], offs[k] + d]
                )
                slab_v[slot, d, pl.ds(k * 16, 16)] = vals

    def fire_wb(h, slot):
        pltpu.async_copy(
            slab_v.at[slot],
            out_hbm.at[h, :, pl.ds(b0, IW)],
            osem.at[slot],
        )

    def wait_wb(slot):
        pltpu.make_async_copy(
            slab_v.at[slot], out_hbm.at[0, :, pl.ds(0, IW)], osem.at[slot]
        ).wait()

    fire_gather(0, 0)
    fire_gather(1, 1)

    def body(g, carry):
        for s in range(2):
            h = 2 * g + s
            wait_gather(s)

            @pl.when(h >= 2)
            def _():
                wait_wb(s)

            transpose(h, s)
            fire_wb(h, s)

            @pl.when(h + 2 < HIST)
            def _():
                fire_gather(h + 2, s)

        return carry

    lax.fori_loop(0, HIST // 2, body, 0)
    wait_wb(0)
    wait_wb(1)


def kernel(inputs, lookup_table):
    table_t = lookup_table.T
    tail = lax.slice(
        lookup_table, (TAIL0, 0), (N_TOKEN, D_EMBED)
    ).reshape(32, DPAD)
    packed = _table_fmt(table_t, tail)
    idx_t = inputs.T
    out5 = _emb_lookup(idx_t, packed)
    emb = jnp.transpose(out5, (2, 0, 1))
    return emb, lookup_table


# speedup vs baseline: 2.2914x; 2.2914x over previous
import functools

import jax
import jax.numpy as jnp
from jax import lax
from jax.experimental import pallas as pl
from jax.experimental.pallas import tpu as pltpu
from jax.experimental.pallas import tpu_sc as plsc

N_TOKEN = 1000000
D_EMBED = 64
BATCH = 4096
HIST = 200

NC = 2
NS = 16
NW = NC * NS
IW = 128
DPAD = 128

W = 256
NBLK = N_TOKEN // W
BLK_MAIN = NBLK // NW * NW
BLKS_PER_W = BLK_MAIN // NW
TAIL0 = NBLK * W

_mesh = plsc.VectorSubcoreMesh(core_axis_name="c", subcore_axis_name="s")
_params = pltpu.CompilerParams(
    use_tc_tiling_on_sc=True, needs_layout_passes=False
)

_LANES = lambda: lax.broadcasted_iota(jnp.int32, (16,), 0)


@functools.partial(
    pl.kernel,
    mesh=_mesh,
    compiler_params=_params,
    out_type=jax.ShapeDtypeStruct((N_TOKEN, DPAD), jnp.float32),
    scratch_types=[
        pltpu.VMEM((2, D_EMBED, W), jnp.float32),
        pltpu.VMEM((2, W, DPAD), jnp.float32),
        pltpu.SemaphoreType.DMA((2,)),
        pltpu.SemaphoreType.DMA((2,)),
    ],
)
def _table_fmt(table_t, tail_slab, out_hbm, sin, sout, isem, wsem):
    wid = lax.axis_index("s") * NC + lax.axis_index("c")

    lanes = _LANES()
    rot = [(lanes + j) & 15 for j in range(16)]

    def col_of(j):
        return wid + NW * j

    def fire_fetch(j, slot):
        pltpu.async_copy(
            table_t.at[:, pl.ds(col_of(j) * W, W)], sin.at[slot],
            isem.at[slot],
        )

    def wait_fetch(slot):
        pltpu.make_async_copy(
            table_t.at[:, pl.ds(0, W)], sin.at[slot], isem.at[slot]
        ).wait()

    def transpose(slot):
        @plsc.parallel_loop(0, (D_EMBED // 16) * (W // 16), unroll=2)
        def _(b):
            d0 = (b & 3) * 16
            c0 = (b >> 2) * 16
            dvec = d0 + lanes
            for j in range(16):
                cvec = c0 + rot[j]
                vals = plsc.load_gather(sin.at[slot], [dvec, cvec])
                plsc.store_scatter(sout.at[slot], [cvec, dvec], vals)

    def fire_wb(j, slot):
        pltpu.async_copy(
            sout.at[slot],
            out_hbm.at[pl.ds(col_of(j) * W, W)],
            wsem.at[slot],
        )

    def wait_wb(slot):
        pltpu.make_async_copy(
            sout.at[slot], out_hbm.at[pl.ds(0, W)], wsem.at[slot]
        ).wait()

    @pl.when(wid == 6)
    def _():
        pltpu.sync_copy(tail_slab, out_hbm.at[pl.ds(TAIL0, N_TOKEN - TAIL0)])

    fire_fetch(0, 0)
    fire_fetch(1, 1)

    def jbody(j, carry):
        s = j % 2
        wait_fetch(s)

        @pl.when(j >= 2)
        def _():
            wait_wb(s)

        transpose(s)
        fire_wb(j, s)

        @pl.when(j + 2 < BLKS_PER_W)
        def _():
            fire_fetch(j + 2, s)

        return carry

    lax.fori_loop(0, BLKS_PER_W, jbody, 0)
    wait_wb(0)
    wait_wb(1)

    for extra in range(NBLK - BLK_MAIN):
        blk = BLK_MAIN + extra

        @pl.when(wid == 8 + extra)
        def _():
            pltpu.async_copy(
                table_t.at[:, pl.ds(blk * W, W)], sin.at[0], isem.at[0]
            )
            wait_fetch(0)
            transpose(0)
            pltpu.async_copy(
                sout.at[0], out_hbm.at[pl.ds(blk * W, W)], wsem.at[0]
            )
            wait_wb(0)


@functools.partial(
    pl.kernel,
    mesh=_mesh,
    compiler_params=_params,
    out_type=jax.ShapeDtypeStruct((HIST, D_EMBED, BATCH), jnp.float32),
    scratch_types=[
        pltpu.VMEM((HIST, IW), jnp.int32),
        pltpu.VMEM((2, IW, DPAD), jnp.float32),
        pltpu.VMEM((2, D_EMBED, IW), jnp.float32),
        pltpu.SemaphoreType.DMA((2,)),
        pltpu.SemaphoreType.DMA((2,)),
    ],
)
def _emb_lookup(idx_t, table_hbm, out_hbm, idx_v, rows_v, slab_v, gsem, osem):
    wid = lax.axis_index("s") * NC + lax.axis_index("c")
    b0 = wid * IW
    pltpu.sync_copy(idx_t.at[:, pl.ds(b0, IW)], idx_v)

    lanes = _LANES()
    rot = [(lanes + j) & 15 for j in range(16)]

    def fire_gather(h, slot):
        pltpu.async_copy(
            table_hbm.at[idx_v.at[h]], rows_v.at[slot], gsem.at[slot]
        )

    def wait_gather(slot):
        pltpu.make_async_copy(
            table_hbm.at[pl.ds(0, IW)], rows_v.at[slot], gsem.at[slot]
        ).wait()

    def transpose(slot):
        @plsc.parallel_loop(0, (D_EMBED // 16) * (IW // 16), unroll=2)
        def _(b):
            d0 = (b & 3) * 16
            c0 = (b >> 2) * 16
            cvec = c0 + lanes
            for j in range(16):
                dvec = d0 + rot[j]
                vals = plsc.load_gather(rows_v.at[slot], [cvec, dvec])
                plsc.store_scatter(slab_v.at[slot], [dvec, cvec], vals)

    def fire_wb(h, slot):
        pltpu.async_copy(
            slab_v.at[slot],
            out_hbm.at[h, :, pl.ds(b0, IW)],
            osem.at[slot],
        )

    def wait_wb(slot):
        pltpu.make_async_copy(
            slab_v.at[slot], out_hbm.at[0, :, pl.ds(0, IW)], osem.at[slot]
        ).wait()

    fire_gather(0, 0)
    fire_gather(1, 1)

    def body(g, carry):
        for s in range(2):
            h = 2 * g + s
            wait_gather(s)

            @pl.when(h >= 2)
            def _():
                wait_wb(s)

            transpose(s)
            fire_wb(h, s)

            @pl.when(h + 2 < HIST)
            def _():
                fire_gather(h + 2, s)

        return carry

    lax.fori_loop(0, HIST // 2, body, 0)
    wait_wb(0)
    wait_wb(1)


def kernel(inputs, lookup_table):
    table_t = lookup_table.T
    tail = jnp.pad(
        lax.slice(lookup_table, (TAIL0, 0), (N_TOKEN, D_EMBED)),
        ((0, 0), (0, DPAD - D_EMBED)),
    )
    padded = _table_fmt(table_t, tail)
    idx_t = inputs.T
    out5 = _emb_lookup(idx_t, padded)
    emb = jnp.transpose(out5, (2, 0, 1))
    return emb, lookup_table


# speedup vs baseline: 2.5327x; 1.1053x over previous
import functools

import jax
import jax.numpy as jnp
from jax import lax
from jax.experimental import pallas as pl
from jax.experimental.pallas import tpu as pltpu
from jax.experimental.pallas import tpu_sc as plsc

N_TOKEN = 1000000
D_EMBED = 64
BATCH = 4096
HIST = 200

NC = 2
NS = 16
NW = NC * NS
IW = 128
DPAD = 128

W = 256
NBLK = N_TOKEN // W
BLK_MAIN = NBLK // NW * NW
BLKS_PER_W = BLK_MAIN // NW
TAIL0 = NBLK * W

_mesh = plsc.VectorSubcoreMesh(core_axis_name="c", subcore_axis_name="s")
_params = pltpu.CompilerParams(
    use_tc_tiling_on_sc=True, needs_layout_passes=False
)

_LANES = lambda: lax.broadcasted_iota(jnp.int32, (16,), 0)


@functools.partial(
    pl.kernel,
    mesh=_mesh,
    compiler_params=_params,
    out_type=(
        jax.ShapeDtypeStruct((N_TOKEN, DPAD), jnp.float32),
        jax.ShapeDtypeStruct((D_EMBED, N_TOKEN), jnp.float32),
    ),
    scratch_types=[
        pltpu.VMEM((2, D_EMBED, W), jnp.float32),
        pltpu.VMEM((2, W, DPAD), jnp.float32),
        pltpu.SemaphoreType.DMA((2,)),
        pltpu.SemaphoreType.DMA((2,)),
        pltpu.SemaphoreType.DMA((2,)),
    ],
)
def _table_fmt(table_t, tail_slab, out_hbm, out2_hbm, sin, sout, isem, wsem,
               w2sem):
    wid = lax.axis_index("s") * NC + lax.axis_index("c")

    lanes = _LANES()
    rot = [(lanes + j) & 15 for j in range(16)]

    def col_of(j):
        return wid + NW * j

    def fire_fetch(j, slot):
        pltpu.async_copy(
            table_t.at[:, pl.ds(col_of(j) * W, W)], sin.at[slot],
            isem.at[slot],
        )

    def wait_fetch(slot):
        pltpu.make_async_copy(
            table_t.at[:, pl.ds(0, W)], sin.at[slot], isem.at[slot]
        ).wait()

    def transpose(slot):
        @plsc.parallel_loop(0, (D_EMBED // 16) * (W // 16), unroll=2)
        def _(b):
            d0 = (b & 3) * 16
            c0 = (b >> 2) * 16
            dvec = d0 + lanes
            for j in range(16):
                cvec = c0 + rot[j]
                vals = plsc.load_gather(sin.at[slot], [dvec, cvec])
                plsc.store_scatter(sout.at[slot], [cvec, dvec], vals)

    def fire_wb(j, slot):
        pltpu.async_copy(
            sout.at[slot],
            out_hbm.at[pl.ds(col_of(j) * W, W)],
            wsem.at[slot],
        )

    def wait_wb(slot):
        pltpu.make_async_copy(
            sout.at[slot], out_hbm.at[pl.ds(0, W)], wsem.at[slot]
        ).wait()

    def fire_wb2(j, slot):
        pltpu.async_copy(
            sin.at[slot],
            out2_hbm.at[:, pl.ds(col_of(j) * W, W)],
            w2sem.at[slot],
        )

    def wait_wb2(slot):
        pltpu.make_async_copy(
            sin.at[slot], out2_hbm.at[:, pl.ds(0, W)], w2sem.at[slot]
        ).wait()

    @pl.when(wid == 6)
    def _():
        pltpu.sync_copy(tail_slab, out_hbm.at[pl.ds(TAIL0, N_TOKEN - TAIL0)])

    fire_fetch(0, 0)
    fire_fetch(1, 1)

    def jbody(j, carry):
        s = j % 2
        wait_fetch(s)
        fire_wb2(j, s)

        @pl.when(j >= 2)
        def _():
            wait_wb(s)

        transpose(s)
        fire_wb(j, s)
        wait_wb2(s)

        @pl.when(j + 2 < BLKS_PER_W)
        def _():
            fire_fetch(j + 2, s)

        return carry

    lax.fori_loop(0, BLKS_PER_W, jbody, 0)
    wait_wb(0)
    wait_wb(1)

    for extra in range(NBLK - BLK_MAIN):
        blk = BLK_MAIN + extra

        @pl.when(wid == 8 + extra)
        def _():
            pltpu.async_copy(
                table_t.at[:, pl.ds(blk * W, W)], sin.at[0], isem.at[0]
            )
            wait_fetch(0)
            pltpu.async_copy(
                sin.at[0], out2_hbm.at[:, pl.ds(blk * W, W)], w2sem.at[0]
            )
            transpose(0)
            pltpu.async_copy(
                sout.at[0], out_hbm.at[pl.ds(blk * W, W)], wsem.at[0]
            )
            wait_wb(0)
            wait_wb2(0)


@functools.partial(
    pl.kernel,
    mesh=_mesh,
    compiler_params=_params,
    out_type=jax.ShapeDtypeStruct((HIST, D_EMBED, BATCH), jnp.float32),
    scratch_types=[
        pltpu.VMEM((HIST, IW), jnp.int32),
        pltpu.VMEM((2, IW, DPAD), jnp.float32),
        pltpu.VMEM((2, D_EMBED, IW), jnp.float32),
        pltpu.SemaphoreType.DMA((2,)),
        pltpu.SemaphoreType.DMA((2,)),
    ],
)
def _emb_lookup(idx_t, table_hbm, out_hbm, idx_v, rows_v, slab_v, gsem, osem):
    wid = lax.axis_index("s") * NC + lax.axis_index("c")
    b0 = wid * IW
    pltpu.sync_copy(idx_t.at[:, pl.ds(b0, IW)], idx_v)

    lanes = _LANES()
    rot = [(lanes + j) & 15 for j in range(16)]

    def fire_gather(h, slot):
        pltpu.async_copy(
            table_hbm.at[idx_v.at[h]], rows_v.at[slot], gsem.at[slot]
        )

    def wait_gather(slot):
        pltpu.make_async_copy(
            table_hbm.at[pl.ds(0, IW)], rows_v.at[slot], gsem.at[slot]
        ).wait()

    def transpose(slot):
        @plsc.parallel_loop(0, (D_EMBED // 16) * (IW // 16), unroll=2)
        def _(b):
            d0 = (b & 3) * 16
            c0 = (b >> 2) * 16
            cvec = c0 + lanes
            for j in range(16):
                dvec = d0 + rot[j]
                vals = plsc.load_gather(rows_v.at[slot], [cvec, dvec])
                plsc.store_scatter(slab_v.at[slot], [dvec, cvec], vals)

    def fire_wb(h, slot):
        pltpu.async_copy(
            slab_v.at[slot],
            out_hbm.at[h, :, pl.ds(b0, IW)],
            osem.at[slot],
        )

    def wait_wb(slot):
        pltpu.make_async_copy(
            slab_v.at[slot], out_hbm.at[0, :, pl.ds(0, IW)], osem.at[slot]
        ).wait()

    fire_gather(0, 0)
    fire_gather(1, 1)

    def body(g, carry):
        for s in range(2):
            h = 2 * g + s
            wait_gather(s)

            @pl.when(h >= 2)
            def _():
                wait_wb(s)

            transpose(s)
            fire_wb(h, s)

            @pl.when(h + 2 < HIST)
            def _():
                fire_gather(h + 2, s)

        return carry

    lax.fori_loop(0, HIST // 2, body, 0)
    wait_wb(0)
    wait_wb(1)


def kernel(inputs, lookup_table):
    table_t = lookup_table.T
    tail = jnp.pad(
        lax.slice(lookup_table, (TAIL0, 0), (N_TOKEN, D_EMBED)),
        ((0, 0), (0, DPAD - D_EMBED)),
    )
    padded, table_echo = _table_fmt(table_t, tail)
    idx_t = inputs.T
    out5 = _emb_lookup(idx_t, padded)
    emb = jnp.transpose(out5, (2, 0, 1))
    table_out = lax.dynamic_update_slice(
        table_echo.T,
        lax.slice(lookup_table, (TAIL0, 0), (N_TOKEN, D_EMBED)),
        (TAIL0, 0),
    )
    return emb, table_out
